# baseline (speedup 1.0000x reference)
import functools

import jax
import jax.numpy as jnp
from jax import lax
from jax.experimental import pallas as pl
from jax.experimental.pallas import tpu as pltpu
from jax.experimental.pallas import tpu_sc as plsc

D_IN, H, D_OUT, K = 768, 16384, 768, 64
T = 256
HB = 1024
NHB = H // HB
N_TOK = 4096
_INF_BITS = 0x7F800000

NW = 32
RPW = N_TOK // NW
CAP = 512
NCB = CAP // 16
VPR = H // 16


def _enc_body(x_ref, beff_ref, we_ref, feat_ref):
    pre = jnp.dot(x_ref[...], we_ref[0], preferred_element_type=jnp.float32)
    feat_ref[...] = jnp.maximum(pre + beff_ref[...], 0.0)


@jax.jit
def _tc_encode(x2, beff, we_r):
    return pl.pallas_call(
        _enc_body,
        grid=(N_TOK // T, NHB),
        in_specs=[
            pl.BlockSpec((T, D_IN), lambda i, j: (i, 0)),
            pl.BlockSpec((1, HB), lambda i, j: (0, j)),
            pl.BlockSpec((1, D_IN, HB), lambda i, j: (j, 0, 0)),
        ],
        out_specs=pl.BlockSpec((T, HB), lambda i, j: (i, j)),
        out_shape=jax.ShapeDtypeStruct((N_TOK, H), jnp.float32),
        compiler_params=pltpu.CompilerParams(
            dimension_semantics=("parallel", "arbitrary"),
        ),
    )(x2, beff, we_r)


def _sc_body(feat_hbm, thr_hbm, rb0, rb1, cand, thrbuf, thref, curref,
             sem0, sem1):
    c_id = lax.axis_index("c")
    s_id = lax.axis_index("s")
    wid = s_id * 2 + c_id
    base = wid * RPW
    iota = lax.iota(jnp.int32, 16)

    def count_gt(t, cur):
        nblk = (cur + 15) // 16

        def cblk(j, c):
            v = cand[pl.ds(j * 16, 16)]
            return c + jnp.sum((v > t).astype(jnp.int32))

        return lax.fori_loop(0, nblk, cblk, jnp.int32(0))

    def bisect(iters, cur):
        lo = lax.bitcast_convert_type(thref[0], jnp.int32)
        hi = jnp.int32(_INF_BITS)

        def it(_, carry):
            lo, hi = carry
            mid = lo + ((hi - lo) >> 1)
            t = lax.bitcast_convert_type(mid, jnp.float32)
            big = count_gt(t, cur) >= K
            return jnp.where(big, mid, lo), jnp.where(big, hi, mid)

        lo, _ = lax.fori_loop(0, iters, it, (lo, hi))
        return lax.bitcast_convert_type(lo, jnp.float32)

    def rebuild():
        cur = curref[0]
        tnew = bisect(12, cur)

        def cblk(j, nc):
            v = cand[pl.ds(j * 16, 16)]
            m = v > tnew
            pos = nc - 1 + plsc.cumsum(m.astype(jnp.int32))
            plsc.store_scatter(cand, [pos], v, mask=m)
            return nc + jnp.sum(m.astype(jnp.int32))

        ncur = lax.fori_loop(0, (cur + 15) // 16, cblk, jnp.int32(0))

        def zblk(j, _):
            v = cand[pl.ds(j * 16, 16)]
            keep = (j * 16 + iota) < ncur
            cand[pl.ds(j * 16, 16)] = jnp.where(keep, v, 0.0)
            return 0

        lax.fori_loop(ncur // 16, (cur + 15) // 16, zblk, 0)
        thref[0] = tnew
        curref[0] = ncur

    def process_row(rb, slot):
        thref[0] = 0.0
        curref[0] = 0
        for j in range(NCB):
            cand[j * 16:(j + 1) * 16] = jnp.zeros((16,), jnp.float32)

        def step(i, _):
            v = rb[pl.ds(i * 16, 16)]
            cnt0 = jnp.sum((v > thref[0]).astype(jnp.int32))

            @pl.when(curref[0] + cnt0 > CAP)
            def _():
                rebuild()

            th = thref[0]
            cur = curref[0]
            m = v > th
            pos = cur - 1 + plsc.cumsum(m.astype(jnp.int32))
            pos = jnp.minimum(pos, CAP - 1)
            plsc.store_scatter(cand, [pos], v, mask=m)
            curref[0] = cur + jnp.sum(m.astype(jnp.int32))
            return 0

        lax.fori_loop(0, VPR, step, 0)
        thr = bisect(31, curref[0])
        plsc.store_scatter(thrbuf, [jnp.full((16,), slot, jnp.int32)],
                           jnp.full((16,), thr, jnp.float32),
                           mask=iota == 0)

    def pair(p, _):
        row_a = base + 2 * p
        pltpu.make_async_copy(feat_hbm.at[row_a], rb0, sem0).wait()
        pltpu.async_copy(feat_hbm.at[row_a + 1], rb1, sem1)
        process_row(rb0, (2 * p) % 16)
        pltpu.make_async_copy(feat_hbm.at[row_a + 1], rb1, sem1).wait()

        @pl.when(p + 1 < RPW // 2)
        def _():
            pltpu.async_copy(feat_hbm.at[row_a + 2], rb0, sem0)

        process_row(rb1, (2 * p + 1) % 16)

        @pl.when((p % 8) == 7)
        def _():
            gbase = base + (p // 8) * 16
            pltpu.sync_copy(thrbuf, thr_hbm.at[pl.ds(gbase, 16)])

        return 0

    pltpu.async_copy(feat_hbm.at[base], rb0, sem0)
    lax.fori_loop(0, RPW // 2, pair, 0)


_sc_select = pl.kernel(
    _sc_body,
    out_type=jax.ShapeDtypeStruct((N_TOK,), jnp.float32),
    mesh=plsc.VectorSubcoreMesh(core_axis_name="c", subcore_axis_name="s"),
    scratch_types=[
        pltpu.VMEM((H,), jnp.float32),
        pltpu.VMEM((H,), jnp.float32),
        pltpu.VMEM((CAP,), jnp.float32),
        pltpu.VMEM((16,), jnp.float32),
        pltpu.SMEM((1,), jnp.float32),
        pltpu.SMEM((1,), jnp.int32),
        pltpu.SemaphoreType.DMA,
        pltpu.SemaphoreType.DMA,
    ],
    compiler_params=pltpu.CompilerParams(needs_layout_passes=False),
)


def _dec_body(feat_ref, thr_ref, wd_ref, bdec_ref, out_ref):
    j = pl.program_id(1)

    @pl.when(j == 0)
    def _():
        out_ref[...] = jnp.broadcast_to(bdec_ref[...], (T, D_OUT))

    fb = feat_ref[...]
    m = jnp.where(fb > thr_ref[...], fb, 0.0).astype(jnp.bfloat16)
    out_ref[...] += jnp.dot(m, wd_ref[0], preferred_element_type=jnp.float32)


@jax.jit
def _tc_decode(feat, thr2, wd_r, bdec2):
    return pl.pallas_call(
        _dec_body,
        grid=(N_TOK // T, NHB),
        in_specs=[
            pl.BlockSpec((T, HB), lambda i, j: (i, j)),
            pl.BlockSpec((T, 1), lambda i, j: (i, 0)),
            pl.BlockSpec((1, HB, D_OUT), lambda i, j: (j, 0, 0)),
            pl.BlockSpec((1, D_OUT), lambda i, j: (0, 0)),
        ],
        out_specs=pl.BlockSpec((T, D_OUT), lambda i, j: (i, 0)),
        out_shape=jax.ShapeDtypeStruct((N_TOK, D_OUT), jnp.float32),
        compiler_params=pltpu.CompilerParams(
            dimension_semantics=("parallel", "arbitrary"),
        ),
    )(feat, thr2, wd_r, bdec2)


def kernel(x, W_enc, b_enc, threshold, W_dec, b_dec):
    B, S, _ = x.shape
    x2 = x.reshape(B * S, D_IN)
    beff = (b_enc - threshold).reshape(1, H)
    bdec2 = b_dec.reshape(1, D_OUT)
    we_r = W_enc.reshape(NHB, HB, D_IN).transpose(0, 2, 1)
    wd_r = W_dec.T.reshape(NHB, HB, D_OUT).astype(jnp.bfloat16)
    feat = _tc_encode(x2, beff, we_r)
    thr = _sc_select(feat)
    out2 = _tc_decode(feat, thr.reshape(N_TOK, 1), wd_r, bdec2)
    return out2.reshape(B, S, D_OUT)


# speedup vs baseline: 2.2021x; 2.2021x over previous
import functools

import jax
import jax.numpy as jnp
from jax import lax
from jax.experimental import pallas as pl
from jax.experimental.pallas import tpu as pltpu
from jax.experimental.pallas import tpu_sc as plsc

D_IN, H, D_OUT, K = 768, 16384, 768, 64
HB = 1024
NHB = H // HB
N_TOK = 4096
_INF_BITS = 0x7F800000

NW = 32
TPW = N_TOK // NW
NGRP = TPW // 16
SEG = 256
NSEG = H // SEG
CHUNK = 128
CAP = 384
TRIG = CAP - CHUNK - 1


def _enc_body(xt_ref, beff_ref, we_ref, feat_ref):
    pre = jnp.dot(we_ref[...], xt_ref[...], preferred_element_type=jnp.float32)
    feat_ref[...] = jnp.maximum(pre + beff_ref[...], 0.0)[None]


@jax.jit
def _tc_encode(xt, beff, W_enc):
    return pl.pallas_call(
        _enc_body,
        grid=(NW, NHB),
        in_specs=[
            pl.BlockSpec((D_IN, TPW), lambda i, j: (0, i)),
            pl.BlockSpec((HB, 1), lambda i, j: (j, 0)),
            pl.BlockSpec((HB, D_IN), lambda i, j: (j, 0)),
        ],
        out_specs=pl.BlockSpec((1, HB, TPW), lambda i, j: (i, j, 0)),
        out_shape=jax.ShapeDtypeStruct((NW, H, TPW), jnp.float32),
        compiler_params=pltpu.CompilerParams(
            dimension_semantics=("parallel", "arbitrary"),
        ),
    )(xt, beff, W_enc)


def _sc_body(feat_hbm, thr_hbm, segA, segB, cand, thrbuf, sem0, sem1):
    c_id = lax.axis_index("c")
    s_id = lax.axis_index("s")
    wid = s_id * 2 + c_id
    base = wid * TPW
    iota = lax.iota(jnp.int32, 16)
    zf = jnp.zeros((16,), jnp.float32)
    zi = jnp.zeros((16,), jnp.int32)

    def seg_copy(s, buf, sem):
        return pltpu.make_async_copy(
            feat_hbm.at[wid, pl.ds(s * SEG, SEG), :], buf, sem)

    def bisect(g, th, curs, iters):
        cbase = g * CAP
        maxc = jnp.max(curs)
        lo = plsc.bitcast(th, jnp.int32)
        hi = jnp.full((16,), _INF_BITS, jnp.int32)

        def bit(_, c2):
            lo, hi = c2
            mid = lo + ((hi - lo) >> 1)
            t = plsc.bitcast(mid, jnp.float32)

            def cnt(s, a):
                vv = cand[pl.ds((cbase + s) * 16, 16)]
                return a + jnp.where(vv > t, 1, 0)

            cv = lax.fori_loop(0, maxc, cnt, zi)
            big = cv >= K
            return jnp.where(big, mid, lo), jnp.where(big, hi, mid)

        lo, _ = lax.fori_loop(0, iters, bit, (lo, hi))
        return lo

    def make_rebuild(g):
        cbase = g * CAP

        def rebuild(th_curs):
            th, curs = th_curs
            maxc = jnp.max(curs)
            tnew = plsc.bitcast(bisect(g, th, curs, 12), jnp.float32)

            def comp(s, nc):
                vv = cand[pl.ds((cbase + s) * 16, 16)]
                cand[pl.ds((cbase + s) * 16, 16)] = zf
                mm = vv > tnew
                plsc.store_scatter(cand, [(cbase + nc) * 16 + iota],
                                   vv, mask=mm)
                return nc + jnp.where(mm, 1, 0)

            ncur = lax.fori_loop(0, maxc, comp, zi)
            return tnew, ncur

        return rebuild

    def make_pass(g, buf):
        cbase = g * CAP
        rebuild = make_rebuild(g)

        def chunk(c, carry2):
            def step(i, c3):
                th, curs = c3
                v = buf[c * CHUNK + i, pl.ds(g * 16, 16)]
                m = v > th
                pos = jnp.minimum((cbase + curs) * 16 + iota,
                                  (cbase + CAP) * 16 - 1)
                plsc.store_scatter(cand, [pos], v, mask=m)
                return th, curs + jnp.where(m, 1, 0)

            th, curs = lax.fori_loop(0, CHUNK, step, carry2)
            return lax.cond(jnp.max(curs) > TRIG, rebuild,
                            lambda tc: tc, (th, curs))

        return chunk

    def zero_cand():
        def zblk(s, x):
            cand[pl.ds(s * 16, 16)] = zf
            return x

        lax.fori_loop(0, NGRP * CAP, zblk, 0)

    def run_seg(buf, states):
        return tuple(
            lax.fori_loop(0, SEG // CHUNK, make_pass(g, buf), states[g])
            for g in range(NGRP))

    zero_cand()
    states = tuple((zf, zi) for _ in range(NGRP))
    seg_copy(0, segA, sem0).start()

    def pair(p, states):
        seg_copy(2 * p, segA, sem0).wait()

        @pl.when(2 * p + 1 < NSEG)
        def _():
            seg_copy(2 * p + 1, segB, sem1).start()

        states = run_seg(segA, states)
        seg_copy(2 * p + 1, segB, sem1).wait()

        @pl.when(2 * p + 2 < NSEG)
        def _():
            seg_copy(2 * p + 2, segA, sem0).start()

        return run_seg(segB, states)

    states = lax.fori_loop(0, NSEG // 2, pair, states)
    for g in range(NGRP):
        th, curs = states[g]
        thrbuf[0:16] = plsc.bitcast(bisect(g, th, curs, 31), jnp.float32)
        pltpu.sync_copy(thrbuf, thr_hbm.at[pl.ds(base + g * 16, 16)])


_sc_select = pl.kernel(
    _sc_body,
    out_type=jax.ShapeDtypeStruct((N_TOK,), jnp.float32),
    mesh=plsc.VectorSubcoreMesh(core_axis_name="c", subcore_axis_name="s"),
    scratch_types=[
        pltpu.VMEM((SEG, TPW), jnp.float32),
        pltpu.VMEM((SEG, TPW), jnp.float32),
        pltpu.VMEM((NGRP * CAP * 16,), jnp.float32),
        pltpu.VMEM((16,), jnp.float32),
        pltpu.SemaphoreType.DMA,
        pltpu.SemaphoreType.DMA,
    ],
    compiler_params=pltpu.CompilerParams(needs_layout_passes=False),
)


def _dec_body(feat_ref, thr_ref, wd_ref, bdec_ref, out_ref):
    j = pl.program_id(1)

    @pl.when(j == 0)
    def _():
        out_ref[...] = jnp.broadcast_to(bdec_ref[...], (TPW, D_OUT))

    ft = feat_ref[0]
    m = jnp.where(ft > thr_ref[0], ft, 0.0).astype(jnp.bfloat16)
    out_ref[...] += lax.dot_general(
        m, wd_ref[0], (((0,), (0,)), ((), ())),
        preferred_element_type=jnp.float32)


@jax.jit
def _tc_decode(feat_g, thr3, wd_r, bdec2):
    return pl.pallas_call(
        _dec_body,
        grid=(NW, NHB),
        in_specs=[
            pl.BlockSpec((1, HB, TPW), lambda i, j: (i, j, 0)),
            pl.BlockSpec((1, 1, TPW), lambda i, j: (i, 0, 0)),
            pl.BlockSpec((1, HB, D_OUT), lambda i, j: (j, 0, 0)),
            pl.BlockSpec((1, D_OUT), lambda i, j: (0, 0)),
        ],
        out_specs=pl.BlockSpec((TPW, D_OUT), lambda i, j: (i, 0)),
        out_shape=jax.ShapeDtypeStruct((N_TOK, D_OUT), jnp.float32),
        compiler_params=pltpu.CompilerParams(
            dimension_semantics=("parallel", "arbitrary"),
        ),
    )(feat_g, thr3, wd_r, bdec2)


def kernel(x, W_enc, b_enc, threshold, W_dec, b_dec):
    B, S, _ = x.shape
    x2 = x.reshape(B * S, D_IN)
    beff = (b_enc - threshold).reshape(H, 1)
    bdec2 = b_dec.reshape(1, D_OUT)
    wd_r = W_dec.T.reshape(NHB, HB, D_OUT).astype(jnp.bfloat16)
    feat_g = _tc_encode(x2.T, beff, W_enc)
    thr = _sc_select(feat_g)
    out2 = _tc_decode(feat_g, thr.reshape(NW, 1, TPW), wd_r, bdec2)
    return out2.reshape(B, S, D_OUT)


# speedup vs baseline: 2.3015x; 1.0451x over previous
import functools

import jax
import jax.numpy as jnp
from jax import lax
from jax.experimental import pallas as pl
from jax.experimental.pallas import tpu as pltpu
from jax.experimental.pallas import tpu_sc as plsc

D_IN, H, D_OUT, K = 768, 16384, 768, 64
HB = 1024
NHB = H // HB
N_TOK = 4096
_INF_BITS = 0x7F800000

NW = 32
TPW = N_TOK // NW
NGRP = TPW // 16
SEG = 256
NSEG = H // SEG
CHUNK = 128
CAP = 384
TRIG = CAP - CHUNK - 1


def _enc_body(xt_ref, beff_ref, we_ref, feat_ref):
    pre = jnp.dot(we_ref[...], xt_ref[...], preferred_element_type=jnp.float32)
    feat_ref[...] = jnp.maximum(pre + beff_ref[...], 0.0)[None]


@jax.jit
def _tc_encode(xt, beff, W_enc):
    return pl.pallas_call(
        _enc_body,
        grid=(NW, NHB),
        in_specs=[
            pl.BlockSpec((D_IN, TPW), lambda i, j: (0, i)),
            pl.BlockSpec((HB, 1), lambda i, j: (j, 0)),
            pl.BlockSpec((HB, D_IN), lambda i, j: (j, 0)),
        ],
        out_specs=pl.BlockSpec((1, HB, TPW), lambda i, j: (i, j, 0)),
        out_shape=jax.ShapeDtypeStruct((NW, H, TPW), jnp.float32),
        compiler_params=pltpu.CompilerParams(
            dimension_semantics=("parallel", "arbitrary"),
        ),
    )(xt, beff, W_enc)


def _sc_body(feat_hbm, thr_hbm, segA, segB, cand, thrbuf, sem0, sem1):
    c_id = lax.axis_index("c")
    s_id = lax.axis_index("s")
    wid = s_id * 2 + c_id
    base = wid * TPW
    iota = lax.iota(jnp.int32, 16)
    zf = jnp.zeros((16,), jnp.float32)
    zi = jnp.zeros((16,), jnp.int32)

    def seg_copy(s, buf, sem):
        return pltpu.make_async_copy(
            feat_hbm.at[wid, pl.ds(s * SEG, SEG), :], buf, sem)

    def bisect(g, th, curs, iters):
        cbase = g * CAP
        maxc = jnp.max(curs)
        lo = plsc.bitcast(th, jnp.int32)
        hi = jnp.full((16,), _INF_BITS, jnp.int32)

        def bit(_, c2):
            lo, hi = c2
            mid = lo + ((hi - lo) >> 1)
            t = plsc.bitcast(mid, jnp.float32)

            def cnt(s, a):
                vv = cand[pl.ds((cbase + s) * 16, 16)]
                return a + jnp.where(vv > t, 1, 0)

            cv = lax.fori_loop(0, maxc, cnt, zi)
            big = cv >= K
            return jnp.where(big, mid, lo), jnp.where(big, hi, mid)

        lo, _ = lax.fori_loop(0, iters, bit, (lo, hi))
        return lo

    def make_rebuild(g):
        cbase = g * CAP

        def rebuild(th_curs):
            th, curs = th_curs
            maxc = jnp.max(curs)
            tnew = plsc.bitcast(bisect(g, th, curs, 12), jnp.float32)

            def comp(s, nc):
                vv = cand[pl.ds((cbase + s) * 16, 16)]
                cand[pl.ds((cbase + s) * 16, 16)] = zf
                mm = vv > tnew
                plsc.store_scatter(cand, [(cbase + nc) * 16 + iota],
                                   vv, mask=mm)
                return nc + jnp.where(mm, 1, 0)

            ncur = lax.fori_loop(0, maxc, comp, zi)
            return tnew, ncur

        return rebuild

    UNROLL = 8

    def _step1(buf, idx, g, th, curs):
        v = buf[idx, pl.ds(g * 16, 16)]
        m = v > th
        pos = jnp.minimum(curs * 16 + (g * CAP * 16 + iota),
                          (g + 1) * CAP * 16 - 1)
        plsc.store_scatter(cand, [pos], v, mask=m)
        return curs + jnp.where(m, 1, 0)

    def make_pass2(ga, gb, buf):
        reb_a = make_rebuild(ga)
        reb_b = make_rebuild(gb)

        def chunk(c, carry2):
            def step(i, c3):
                (tha, cua), (thb, cub) = c3
                for u in range(UNROLL):
                    idx = c * CHUNK + i * UNROLL + u
                    cua = _step1(buf, idx, ga, tha, cua)
                    cub = _step1(buf, idx, gb, thb, cub)
                return (tha, cua), (thb, cub)

            sa, sb = lax.fori_loop(0, CHUNK // UNROLL, step, carry2)
            sa = lax.cond(jnp.max(sa[1]) > TRIG, reb_a, lambda tc: tc, sa)
            sb = lax.cond(jnp.max(sb[1]) > TRIG, reb_b, lambda tc: tc, sb)
            return sa, sb

        return chunk

    def zero_cand():
        def zblk(s, x):
            cand[pl.ds(s * 16, 16)] = zf
            return x

        lax.fori_loop(0, NGRP * CAP, zblk, 0)

    def run_seg(buf, states):
        out = list(states)
        for ga in range(0, NGRP, 2):
            sa, sb = lax.fori_loop(0, SEG // CHUNK, make_pass2(ga, ga + 1, buf),
                                   (states[ga], states[ga + 1]))
            out[ga], out[ga + 1] = sa, sb
        return tuple(out)

    zero_cand()
    states = tuple((zf, zi) for _ in range(NGRP))
    seg_copy(0, segA, sem0).start()

    def pair(p, states):
        seg_copy(2 * p, segA, sem0).wait()

        @pl.when(2 * p + 1 < NSEG)
        def _():
            seg_copy(2 * p + 1, segB, sem1).start()

        states = run_seg(segA, states)
        seg_copy(2 * p + 1, segB, sem1).wait()

        @pl.when(2 * p + 2 < NSEG)
        def _():
            seg_copy(2 * p + 2, segA, sem0).start()

        return run_seg(segB, states)

    states = lax.fori_loop(0, NSEG // 2, pair, states)
    for g in range(NGRP):
        th, curs = states[g]
        thrbuf[0:16] = plsc.bitcast(bisect(g, th, curs, 31), jnp.float32)
        pltpu.sync_copy(thrbuf, thr_hbm.at[pl.ds(base + g * 16, 16)])


_sc_select = pl.kernel(
    _sc_body,
    out_type=jax.ShapeDtypeStruct((N_TOK,), jnp.float32),
    mesh=plsc.VectorSubcoreMesh(core_axis_name="c", subcore_axis_name="s"),
    scratch_types=[
        pltpu.VMEM((SEG, TPW), jnp.float32),
        pltpu.VMEM((SEG, TPW), jnp.float32),
        pltpu.VMEM((NGRP * CAP * 16,), jnp.float32),
        pltpu.VMEM((16,), jnp.float32),
        pltpu.SemaphoreType.DMA,
        pltpu.SemaphoreType.DMA,
    ],
    compiler_params=pltpu.CompilerParams(needs_layout_passes=False),
)


def _dec_body(feat_ref, thr_ref, wd_ref, bdec_ref, out_ref):
    j = pl.program_id(1)

    @pl.when(j == 0)
    def _():
        out_ref[...] = jnp.broadcast_to(bdec_ref[...], (TPW, D_OUT))

    ft = feat_ref[0]
    m = jnp.where(ft > thr_ref[0], ft, 0.0).astype(jnp.bfloat16)
    out_ref[...] += lax.dot_general(
        m, wd_ref[0], (((0,), (0,)), ((), ())),
        preferred_element_type=jnp.float32)


@jax.jit
def _tc_decode(feat_g, thr3, wd_r, bdec2):
    return pl.pallas_call(
        _dec_body,
        grid=(NW, NHB),
        in_specs=[
            pl.BlockSpec((1, HB, TPW), lambda i, j: (i, j, 0)),
            pl.BlockSpec((1, 1, TPW), lambda i, j: (i, 0, 0)),
            pl.BlockSpec((1, HB, D_OUT), lambda i, j: (j, 0, 0)),
            pl.BlockSpec((1, D_OUT), lambda i, j: (0, 0)),
        ],
        out_specs=pl.BlockSpec((TPW, D_OUT), lambda i, j: (i, 0)),
        out_shape=jax.ShapeDtypeStruct((N_TOK, D_OUT), jnp.float32),
        compiler_params=pltpu.CompilerParams(
            dimension_semantics=("parallel", "arbitrary"),
        ),
    )(feat_g, thr3, wd_r, bdec2)


def kernel(x, W_enc, b_enc, threshold, W_dec, b_dec):
    B, S, _ = x.shape
    x2 = x.reshape(B * S, D_IN)
    beff = (b_enc - threshold).reshape(H, 1)
    bdec2 = b_dec.reshape(1, D_OUT)
    wd_r = W_dec.T.reshape(NHB, HB, D_OUT).astype(jnp.bfloat16)
    feat_g = _tc_encode(x2.T, beff, W_enc)
    thr = _sc_select(feat_g)
    out2 = _tc_decode(feat_g, thr.reshape(NW, 1, TPW), wd_r, bdec2)
    return out2.reshape(B, S, D_OUT)


# speedup vs baseline: 2.7862x; 1.2106x over previous
import functools

import jax
import jax.numpy as jnp
from jax import lax
from jax.experimental import pallas as pl
from jax.experimental.pallas import tpu as pltpu
from jax.experimental.pallas import tpu_sc as plsc

D_IN, H, D_OUT, K = 768, 16384, 768, 64
HB = 1024
NHB = H // HB
N_TOK = 4096
_INF_BITS = 0x7F800000

NW = 32
TPW = N_TOK // NW
NGRP = TPW // 16
SEG = 256
NSEG = H // SEG
CHUNK = 128
CAP = 384
TRIG = CAP - CHUNK - 1


def _enc_body(xt_ref, beff_ref, we_ref, feat_ref):
    pre = jnp.dot(we_ref[...], xt_ref[...], preferred_element_type=jnp.float32)
    feat_ref[...] = jnp.maximum(pre + beff_ref[...], 0.0)[None]


@jax.jit
def _tc_encode(xt, beff, W_enc):
    return pl.pallas_call(
        _enc_body,
        grid=(NW, NHB),
        in_specs=[
            pl.BlockSpec((D_IN, TPW), lambda i, j: (0, i)),
            pl.BlockSpec((HB, 1), lambda i, j: (j, 0)),
            pl.BlockSpec((HB, D_IN), lambda i, j: (j, 0)),
        ],
        out_specs=pl.BlockSpec((1, HB, TPW), lambda i, j: (i, j, 0)),
        out_shape=jax.ShapeDtypeStruct((NW, H, TPW), jnp.float32),
        compiler_params=pltpu.CompilerParams(
            dimension_semantics=("parallel", "arbitrary"),
        ),
    )(xt, beff, W_enc)


def _sc_body(feat_hbm, thr_hbm, segA, segB, cand, thrbuf, sem0, sem1):
    c_id = lax.axis_index("c")
    s_id = lax.axis_index("s")
    wid = s_id * 2 + c_id
    base = wid * TPW
    iota = lax.iota(jnp.int32, 16)
    zf = jnp.zeros((16,), jnp.float32)
    zi = jnp.zeros((16,), jnp.int32)

    def seg_copy(s, buf, sem):
        return pltpu.make_async_copy(
            feat_hbm.at[wid, pl.ds(s * SEG, SEG), :], buf, sem)

    def bisect(g, th, curs, iters):
        cbase = g * CAP
        maxc = jnp.max(curs)
        lo = plsc.bitcast(th, jnp.int32)
        hi = jnp.full((16,), _INF_BITS, jnp.int32)

        nblk8 = (maxc + 7) >> 3

        def bit(_, c2):
            lo, hi = c2
            mid = lo + ((hi - lo) >> 1)
            t = plsc.bitcast(mid, jnp.float32)

            def cnt(s, accs):
                a0, a1 = accs
                for u in range(8):
                    vv = cand[pl.ds((cbase + s * 8 + u) * 16, 16)]
                    w = jnp.where(vv > t, 1, 0)
                    if u % 2 == 0:
                        a0 = a0 + w
                    else:
                        a1 = a1 + w
                return a0, a1

            a0, a1 = lax.fori_loop(0, nblk8, cnt, (zi, zi))
            big = (a0 + a1) >= K
            return jnp.where(big, mid, lo), jnp.where(big, hi, mid)

        lo, _ = lax.fori_loop(0, iters, bit, (lo, hi))
        return lo

    def make_rebuild(g):
        cbase = g * CAP

        def rebuild(th_curs):
            th, curs = th_curs
            maxc = jnp.max(curs)
            tnew = plsc.bitcast(bisect(g, th, curs, 12), jnp.float32)

            def comp(s, nc):
                for u in range(4):
                    vv = cand[pl.ds((cbase + s * 4 + u) * 16, 16)]
                    cand[pl.ds((cbase + s * 4 + u) * 16, 16)] = zf
                    mm = vv > tnew
                    plsc.store_scatter(cand, [(cbase + nc) * 16 + iota],
                                       vv, mask=mm)
                    nc = nc + jnp.where(mm, 1, 0)
                return nc

            ncur = lax.fori_loop(0, (maxc + 3) >> 2, comp, zi)
            return tnew, ncur

        return rebuild

    UNROLL = 8

    def _step1(buf, idx, g, th, curs):
        v = buf[idx, pl.ds(g * 16, 16)]
        m = v > th
        pos = jnp.minimum(curs * 16 + (g * CAP * 16 + iota),
                          (g + 1) * CAP * 16 - 1)
        plsc.store_scatter(cand, [pos], v, mask=m)
        return curs + jnp.where(m, 1, 0)

    def make_pass2(ga, gb, buf):
        reb_a = make_rebuild(ga)
        reb_b = make_rebuild(gb)

        def chunk(c, carry2):
            def step(i, c3):
                (tha, cua), (thb, cub) = c3
                for u in range(UNROLL):
                    idx = c * CHUNK + i * UNROLL + u
                    cua = _step1(buf, idx, ga, tha, cua)
                    cub = _step1(buf, idx, gb, thb, cub)
                return (tha, cua), (thb, cub)

            sa, sb = lax.fori_loop(0, CHUNK // UNROLL, step, carry2)
            sa = lax.cond(jnp.max(sa[1]) > TRIG, reb_a, lambda tc: tc, sa)
            sb = lax.cond(jnp.max(sb[1]) > TRIG, reb_b, lambda tc: tc, sb)
            return sa, sb

        return chunk

    def zero_cand():
        def zblk(s, x):
            cand[pl.ds(s * 16, 16)] = zf
            return x

        lax.fori_loop(0, NGRP * CAP, zblk, 0)

    def run_seg(buf, states):
        out = list(states)
        for ga in range(0, NGRP, 2):
            sa, sb = lax.fori_loop(0, SEG // CHUNK, make_pass2(ga, ga + 1, buf),
                                   (states[ga], states[ga + 1]))
            out[ga], out[ga + 1] = sa, sb
        return tuple(out)

    zero_cand()
    states = tuple((zf, zi) for _ in range(NGRP))
    seg_copy(0, segA, sem0).start()

    def pair(p, states):
        seg_copy(2 * p, segA, sem0).wait()

        @pl.when(2 * p + 1 < NSEG)
        def _():
            seg_copy(2 * p + 1, segB, sem1).start()

        states = run_seg(segA, states)
        seg_copy(2 * p + 1, segB, sem1).wait()

        @pl.when(2 * p + 2 < NSEG)
        def _():
            seg_copy(2 * p + 2, segA, sem0).start()

        return run_seg(segB, states)

    states = lax.fori_loop(0, NSEG // 2, pair, states)
    for g in range(NGRP):
        th, curs = states[g]
        thrbuf[0:16] = plsc.bitcast(bisect(g, th, curs, 31), jnp.float32)
        pltpu.sync_copy(thrbuf, thr_hbm.at[pl.ds(base + g * 16, 16)])


_sc_select = pl.kernel(
    _sc_body,
    out_type=jax.ShapeDtypeStruct((N_TOK,), jnp.float32),
    mesh=plsc.VectorSubcoreMesh(core_axis_name="c", subcore_axis_name="s"),
    scratch_types=[
        pltpu.VMEM((SEG, TPW), jnp.float32),
        pltpu.VMEM((SEG, TPW), jnp.float32),
        pltpu.VMEM((NGRP * CAP * 16,), jnp.float32),
        pltpu.VMEM((16,), jnp.float32),
        pltpu.SemaphoreType.DMA,
        pltpu.SemaphoreType.DMA,
    ],
    compiler_params=pltpu.CompilerParams(needs_layout_passes=False),
)


def _dec_body(feat_ref, thr_ref, wd_ref, bdec_ref, out_ref):
    j = pl.program_id(1)

    @pl.when(j == 0)
    def _():
        out_ref[...] = jnp.broadcast_to(bdec_ref[...], (TPW, D_OUT))

    ft = feat_ref[0]
    m = jnp.where(ft > thr_ref[0], ft, 0.0).astype(jnp.bfloat16)
    out_ref[...] += lax.dot_general(
        m, wd_ref[0], (((0,), (0,)), ((), ())),
        preferred_element_type=jnp.float32)


@jax.jit
def _tc_decode(feat_g, thr3, wd_r, bdec2):
    return pl.pallas_call(
        _dec_body,
        grid=(NW, NHB),
        in_specs=[
            pl.BlockSpec((1, HB, TPW), lambda i, j: (i, j, 0)),
            pl.BlockSpec((1, 1, TPW), lambda i, j: (i, 0, 0)),
            pl.BlockSpec((1, HB, D_OUT), lambda i, j: (j, 0, 0)),
            pl.BlockSpec((1, D_OUT), lambda i, j: (0, 0)),
        ],
        out_specs=pl.BlockSpec((TPW, D_OUT), lambda i, j: (i, 0)),
        out_shape=jax.ShapeDtypeStruct((N_TOK, D_OUT), jnp.float32),
        compiler_params=pltpu.CompilerParams(
            dimension_semantics=("parallel", "arbitrary"),
        ),
    )(feat_g, thr3, wd_r, bdec2)


def kernel(x, W_enc, b_enc, threshold, W_dec, b_dec):
    B, S, _ = x.shape
    x2 = x.reshape(B * S, D_IN)
    beff = (b_enc - threshold).reshape(H, 1)
    bdec2 = b_dec.reshape(1, D_OUT)
    wd_r = W_dec.T.reshape(NHB, HB, D_OUT).astype(jnp.bfloat16)
    feat_g = _tc_encode(x2.T, beff, W_enc)
    thr = _sc_select(feat_g)
    out2 = _tc_decode(feat_g, thr.reshape(NW, 1, TPW), wd_r, bdec2)
    return out2.reshape(B, S, D_OUT)


# speedup vs baseline: 3.1099x; 1.1162x over previous
import functools

import jax
import jax.numpy as jnp
from jax import lax
from jax.experimental import pallas as pl
from jax.experimental.pallas import tpu as pltpu
from jax.experimental.pallas import tpu_sc as plsc

D_IN, H, D_OUT, K = 768, 16384, 768, 64
HB = 1024
NHB = H // HB
N_TOK = 4096
_INF_BITS = 0x7F800000

NW = 32
TPW = N_TOK // NW
NGRP = TPW // 16
SEG = 256
NSEG = H // SEG
CHUNK = 128
CAP = 384
TRIG = CAP - CHUNK - 1


def _enc_body(xt_ref, beff_ref, we_ref, feat_ref):
    pre = jnp.dot(we_ref[...], xt_ref[...], preferred_element_type=jnp.float32)
    feat_ref[...] = jnp.maximum(pre + beff_ref[...], 0.0)[None]


@jax.jit
def _tc_encode(xt, beff, W_enc):
    return pl.pallas_call(
        _enc_body,
        grid=(NW, NHB),
        in_specs=[
            pl.BlockSpec((D_IN, TPW), lambda i, j: (0, i)),
            pl.BlockSpec((HB, 1), lambda i, j: (j, 0)),
            pl.BlockSpec((HB, D_IN), lambda i, j: (j, 0)),
        ],
        out_specs=pl.BlockSpec((1, HB, TPW), lambda i, j: (i, j, 0)),
        out_shape=jax.ShapeDtypeStruct((NW, H, TPW), jnp.float32),
        compiler_params=pltpu.CompilerParams(
            dimension_semantics=("parallel", "arbitrary"),
        ),
    )(xt, beff, W_enc)


def _sc_body(feat_hbm, thr_hbm, segA, segB, cand, thrbuf, sem0, sem1):
    c_id = lax.axis_index("c")
    s_id = lax.axis_index("s")
    wid = s_id * 2 + c_id
    base = wid * TPW
    iota = lax.iota(jnp.int32, 16)
    zf = jnp.zeros((16,), jnp.float32)
    zi = jnp.zeros((16,), jnp.int32)

    def seg_copy(s, buf, sem):
        return pltpu.make_async_copy(
            feat_hbm.at[wid, pl.ds(s * SEG, SEG), :], buf, sem)

    def bisect(g, th, curs, iters):
        cbase = g * CAP
        maxc = jnp.max(curs)
        lo = plsc.bitcast(th, jnp.int32)
        hi = jnp.full((16,), _INF_BITS, jnp.int32)

        nblk8 = (maxc + 7) >> 3

        def bit(_, c2):
            lo, hi = c2
            mid = lo + ((hi - lo) >> 1)
            t = plsc.bitcast(mid, jnp.float32)

            def cnt(s, accs):
                a0, a1 = accs
                for u in range(8):
                    vv = cand[pl.ds((cbase + s * 8 + u) * 16, 16)]
                    w = jnp.where(vv > t, 1, 0)
                    if u % 2 == 0:
                        a0 = a0 + w
                    else:
                        a1 = a1 + w
                return a0, a1

            a0, a1 = lax.fori_loop(0, nblk8, cnt, (zi, zi))
            big = (a0 + a1) >= K
            return jnp.where(big, mid, lo), jnp.where(big, hi, mid)

        lo, _ = lax.fori_loop(0, iters, bit, (lo, hi))
        return lo

    def make_rebuild(g):
        cbase = g * CAP

        def rebuild(th_curs):
            th, curs = th_curs
            maxc = jnp.max(curs)
            tnew = plsc.bitcast(bisect(g, th, curs, 12), jnp.float32)

            def comp(s, nc):
                for u in range(4):
                    vv = cand[pl.ds((cbase + s * 4 + u) * 16, 16)]
                    cand[pl.ds((cbase + s * 4 + u) * 16, 16)] = zf
                    mm = vv > tnew
                    plsc.store_scatter(cand, [(cbase + nc) * 16 + iota],
                                       vv, mask=mm)
                    nc = nc + jnp.where(mm, 1, 0)
                return nc

            ncur = lax.fori_loop(0, (maxc + 3) >> 2, comp, zi)
            return tnew, ncur

        return rebuild

    UNROLL = 8

    def _step2(buf, idx, g, th, curs):
        v1 = buf[idx, pl.ds(g * 16, 16)]
        v2 = buf[idx + 1, pl.ds(g * 16, 16)]
        m1 = v1 > th
        m2 = v2 > th
        w1 = jnp.where(m1, 1, 0)
        w2 = jnp.where(m2, 1, 0)
        lim = (g + 1) * CAP * 16 - 1
        base = g * CAP * 16 + iota
        pos1 = jnp.minimum(curs * 16 + base, lim)
        pos2 = jnp.minimum((curs + w1) * 16 + base, lim)
        plsc.store_scatter(cand, [pos1], v1, mask=m1)
        plsc.store_scatter(cand, [pos2], v2, mask=m2)
        return curs + (w1 + w2)

    def make_pass2(ga, gb, buf):
        reb_a = make_rebuild(ga)
        reb_b = make_rebuild(gb)

        def chunk(c, carry2):
            def step(i, c3):
                (tha, cua), (thb, cub) = c3
                for u in range(UNROLL // 2):
                    idx = c * CHUNK + i * UNROLL + 2 * u
                    cua = _step2(buf, idx, ga, tha, cua)
                    cub = _step2(buf, idx, gb, thb, cub)
                return (tha, cua), (thb, cub)

            sa, sb = lax.fori_loop(0, CHUNK // UNROLL, step, carry2)
            sa = lax.cond(jnp.max(sa[1]) > TRIG, reb_a, lambda tc: tc, sa)
            sb = lax.cond(jnp.max(sb[1]) > TRIG, reb_b, lambda tc: tc, sb)
            return sa, sb

        return chunk

    def zero_cand():
        def zblk(s, x):
            cand[pl.ds(s * 16, 16)] = zf
            return x

        lax.fori_loop(0, NGRP * CAP, zblk, 0)

    def run_seg(buf, states):
        out = list(states)
        for ga in range(0, NGRP, 2):
            sa, sb = lax.fori_loop(0, SEG // CHUNK, make_pass2(ga, ga + 1, buf),
                                   (states[ga], states[ga + 1]))
            out[ga], out[ga + 1] = sa, sb
        return tuple(out)

    zero_cand()
    states = tuple((zf, zi) for _ in range(NGRP))
    seg_copy(0, segA, sem0).start()

    def pair(p, states):
        seg_copy(2 * p, segA, sem0).wait()

        @pl.when(2 * p + 1 < NSEG)
        def _():
            seg_copy(2 * p + 1, segB, sem1).start()

        states = run_seg(segA, states)
        seg_copy(2 * p + 1, segB, sem1).wait()

        @pl.when(2 * p + 2 < NSEG)
        def _():
            seg_copy(2 * p + 2, segA, sem0).start()

        return run_seg(segB, states)

    states = lax.fori_loop(0, NSEG // 2, pair, states)
    for g in range(NGRP):
        th, curs = states[g]
        thrbuf[0:16] = plsc.bitcast(bisect(g, th, curs, 31), jnp.float32)
        pltpu.sync_copy(thrbuf, thr_hbm.at[pl.ds(base + g * 16, 16)])


_sc_select = pl.kernel(
    _sc_body,
    out_type=jax.ShapeDtypeStruct((N_TOK,), jnp.float32),
    mesh=plsc.VectorSubcoreMesh(core_axis_name="c", subcore_axis_name="s"),
    scratch_types=[
        pltpu.VMEM((SEG, TPW), jnp.float32),
        pltpu.VMEM((SEG, TPW), jnp.float32),
        pltpu.VMEM((NGRP * CAP * 16,), jnp.float32),
        pltpu.VMEM((16,), jnp.float32),
        pltpu.SemaphoreType.DMA,
        pltpu.SemaphoreType.DMA,
    ],
    compiler_params=pltpu.CompilerParams(needs_layout_passes=False),
)


def _dec_body(feat_ref, thr_ref, wd_ref, bdec_ref, out_ref):
    j = pl.program_id(1)

    @pl.when(j == 0)
    def _():
        out_ref[...] = jnp.broadcast_to(bdec_ref[...], (TPW, D_OUT))

    ft = feat_ref[0]
    m = jnp.where(ft > thr_ref[0], ft, 0.0).astype(jnp.bfloat16)
    out_ref[...] += lax.dot_general(
        m, wd_ref[0], (((0,), (0,)), ((), ())),
        preferred_element_type=jnp.float32)


@jax.jit
def _tc_decode(feat_g, thr3, wd_r, bdec2):
    return pl.pallas_call(
        _dec_body,
        grid=(NW, NHB),
        in_specs=[
            pl.BlockSpec((1, HB, TPW), lambda i, j: (i, j, 0)),
            pl.BlockSpec((1, 1, TPW), lambda i, j: (i, 0, 0)),
            pl.BlockSpec((1, HB, D_OUT), lambda i, j: (j, 0, 0)),
            pl.BlockSpec((1, D_OUT), lambda i, j: (0, 0)),
        ],
        out_specs=pl.BlockSpec((TPW, D_OUT), lambda i, j: (i, 0)),
        out_shape=jax.ShapeDtypeStruct((N_TOK, D_OUT), jnp.float32),
        compiler_params=pltpu.CompilerParams(
            dimension_semantics=("parallel", "arbitrary"),
        ),
    )(feat_g, thr3, wd_r, bdec2)


def kernel(x, W_enc, b_enc, threshold, W_dec, b_dec):
    B, S, _ = x.shape
    x2 = x.reshape(B * S, D_IN)
    beff = (b_enc - threshold).reshape(H, 1)
    bdec2 = b_dec.reshape(1, D_OUT)
    wd_r = W_dec.T.reshape(NHB, HB, D_OUT).astype(jnp.bfloat16)
    feat_g = _tc_encode(x2.T, beff, W_enc)
    thr = _sc_select(feat_g)
    out2 = _tc_decode(feat_g, thr.reshape(NW, 1, TPW), wd_r, bdec2)
    return out2.reshape(B, S, D_OUT)


# speedup vs baseline: 3.8900x; 1.2509x over previous
import functools

import jax
import jax.numpy as jnp
from jax import lax
from jax.experimental import pallas as pl
from jax.experimental.pallas import tpu as pltpu
from jax.experimental.pallas import tpu_sc as plsc

D_IN, H, D_OUT, K = 768, 16384, 768, 64
HB = 1024
NHB = H // HB
N_TOK = 4096
_INF_BITS = 0x7F800000

NW = 32
TPW = N_TOK // NW
NGRP = TPW // 16
SEG = 256
NSEG = H // SEG
CHUNK = 128
CAP = 384
TRIG = CAP - CHUNK - 1


def _enc_body(xt_ref, beff_ref, we_ref, feat_ref):
    pre = jnp.dot(we_ref[...], xt_ref[...], preferred_element_type=jnp.float32)
    feat_ref[...] = jnp.maximum(pre + beff_ref[...], 0.0)[None]


@jax.jit
def _tc_encode(xt, beff, W_enc):
    return pl.pallas_call(
        _enc_body,
        grid=(NW, NHB),
        in_specs=[
            pl.BlockSpec((D_IN, TPW), lambda i, j: (0, i)),
            pl.BlockSpec((HB, 1), lambda i, j: (j, 0)),
            pl.BlockSpec((HB, D_IN), lambda i, j: (j, 0)),
        ],
        out_specs=pl.BlockSpec((1, HB, TPW), lambda i, j: (i, j, 0)),
        out_shape=jax.ShapeDtypeStruct((NW, H, TPW), jnp.float32),
        compiler_params=pltpu.CompilerParams(
            dimension_semantics=("parallel", "arbitrary"),
        ),
    )(xt, beff, W_enc)


def _sc_body(feat_hbm, thr_hbm, segA, segB, cand, thrbuf, sem0, sem1):
    c_id = lax.axis_index("c")
    s_id = lax.axis_index("s")
    wid = s_id * 2 + c_id
    base = wid * TPW
    iota = lax.iota(jnp.int32, 16)
    zf = jnp.zeros((16,), jnp.float32)
    zi = jnp.zeros((16,), jnp.int32)

    def seg_copy(s, buf, sem):
        return pltpu.make_async_copy(
            feat_hbm.at[wid, pl.ds(s * SEG, SEG), :], buf, sem)

    def bisect(g, th, curs, iters):
        cbase = g * CAP
        maxc = jnp.max(curs)
        lo = plsc.bitcast(th, jnp.int32)
        hi = jnp.full((16,), _INF_BITS, jnp.int32)

        nblk8 = (maxc + 7) >> 3

        def bit(_, c2):
            lo, hi = c2
            mid = lo + ((hi - lo) >> 1)
            t = plsc.bitcast(mid, jnp.float32)

            def cnt(s, accs):
                a0, a1 = accs
                for u in range(8):
                    vv = cand[pl.ds((cbase + s * 8 + u) * 16, 16)]
                    w = jnp.where(vv > t, 1, 0)
                    if u % 2 == 0:
                        a0 = a0 + w
                    else:
                        a1 = a1 + w
                return a0, a1

            a0, a1 = lax.fori_loop(0, nblk8, cnt, (zi, zi))
            big = (a0 + a1) >= K
            return jnp.where(big, mid, lo), jnp.where(big, hi, mid)

        lo, _ = lax.fori_loop(0, iters, bit, (lo, hi))
        return lo

    def make_rebuild(g):
        cbase = g * CAP

        def rebuild(th_curs):
            th, curs = th_curs
            maxc = jnp.max(curs)
            tnew = plsc.bitcast(bisect(g, th, curs, 12), jnp.float32)

            def comp(s, nc):
                for u in range(4):
                    vv = cand[pl.ds((cbase + s * 4 + u) * 16, 16)]
                    cand[pl.ds((cbase + s * 4 + u) * 16, 16)] = zf
                    mm = vv > tnew
                    plsc.store_scatter(cand, [(cbase + nc) * 16 + iota],
                                       vv, mask=mm)
                    nc = nc + jnp.where(mm, 1, 0)
                return nc

            ncur = lax.fori_loop(0, (maxc + 3) >> 2, comp, zi)
            return tnew, ncur

        return rebuild

    UNROLL = 8

    def _step2(buf, idx, g, th, curs):
        v1 = buf[idx, pl.ds(g * 16, 16)]
        v2 = buf[idx + 1, pl.ds(g * 16, 16)]
        m1 = v1 > th
        m2 = v2 > th
        w1 = jnp.where(m1, 1, 0)
        w2 = jnp.where(m2, 1, 0)
        lim = (g + 1) * CAP * 16 - 1
        base = g * CAP * 16 + iota
        pos1 = jnp.minimum(curs * 16 + base, lim)
        pos2 = jnp.minimum((curs + w1) * 16 + base, lim)
        plsc.store_scatter(cand, [pos1], v1, mask=m1)
        plsc.store_scatter(cand, [pos2], v2, mask=m2)
        return curs + (w1 + w2)

    def make_pass2(ga, gb, buf):
        reb_a = make_rebuild(ga)
        reb_b = make_rebuild(gb)

        def chunk(c, carry2):
            def step(i, c3):
                (tha, cua), (thb, cub) = c3
                idx = c * CHUNK + i
                cua = _step2(buf, idx, ga, tha, cua)
                cub = _step2(buf, idx, gb, thb, cub)
                return (tha, cua), (thb, cub)

            sa, sb = plsc.parallel_loop(
                0, CHUNK, step=2, unroll=UNROLL // 2, carry=carry2)(step)
            sa = lax.cond(jnp.max(sa[1]) > TRIG, reb_a, lambda tc: tc, sa)
            sb = lax.cond(jnp.max(sb[1]) > TRIG, reb_b, lambda tc: tc, sb)
            return sa, sb

        return chunk

    def zero_cand():
        def zblk(s, x):
            cand[pl.ds(s * 16, 16)] = zf
            return x

        lax.fori_loop(0, NGRP * CAP, zblk, 0)

    def run_seg(buf, states):
        out = list(states)
        for ga in range(0, NGRP, 2):
            sa, sb = lax.fori_loop(0, SEG // CHUNK, make_pass2(ga, ga + 1, buf),
                                   (states[ga], states[ga + 1]))
            out[ga], out[ga + 1] = sa, sb
        return tuple(out)

    zero_cand()
    states = tuple((zf, zi) for _ in range(NGRP))
    seg_copy(0, segA, sem0).start()

    def pair(p, states):
        seg_copy(2 * p, segA, sem0).wait()

        @pl.when(2 * p + 1 < NSEG)
        def _():
            seg_copy(2 * p + 1, segB, sem1).start()

        states = run_seg(segA, states)
        seg_copy(2 * p + 1, segB, sem1).wait()

        @pl.when(2 * p + 2 < NSEG)
        def _():
            seg_copy(2 * p + 2, segA, sem0).start()

        return run_seg(segB, states)

    states = lax.fori_loop(0, NSEG // 2, pair, states)
    for g in range(NGRP):
        th, curs = states[g]
        thrbuf[0:16] = plsc.bitcast(bisect(g, th, curs, 31), jnp.float32)
        pltpu.sync_copy(thrbuf, thr_hbm.at[pl.ds(base + g * 16, 16)])


_sc_select = pl.kernel(
    _sc_body,
    out_type=jax.ShapeDtypeStruct((N_TOK,), jnp.float32),
    mesh=plsc.VectorSubcoreMesh(core_axis_name="c", subcore_axis_name="s"),
    scratch_types=[
        pltpu.VMEM((SEG, TPW), jnp.float32),
        pltpu.VMEM((SEG, TPW), jnp.float32),
        pltpu.VMEM((NGRP * CAP * 16,), jnp.float32),
        pltpu.VMEM((16,), jnp.float32),
        pltpu.SemaphoreType.DMA,
        pltpu.SemaphoreType.DMA,
    ],
    compiler_params=pltpu.CompilerParams(needs_layout_passes=False),
)


def _dec_body(feat_ref, thr_ref, wd_ref, bdec_ref, out_ref):
    j = pl.program_id(1)

    @pl.when(j == 0)
    def _():
        out_ref[...] = jnp.broadcast_to(bdec_ref[...], (TPW, D_OUT))

    ft = feat_ref[0]
    m = jnp.where(ft > thr_ref[0], ft, 0.0).astype(jnp.bfloat16)
    out_ref[...] += lax.dot_general(
        m, wd_ref[0], (((0,), (0,)), ((), ())),
        preferred_element_type=jnp.float32)


@jax.jit
def _tc_decode(feat_g, thr3, wd_r, bdec2):
    return pl.pallas_call(
        _dec_body,
        grid=(NW, NHB),
        in_specs=[
            pl.BlockSpec((1, HB, TPW), lambda i, j: (i, j, 0)),
            pl.BlockSpec((1, 1, TPW), lambda i, j: (i, 0, 0)),
            pl.BlockSpec((1, HB, D_OUT), lambda i, j: (j, 0, 0)),
            pl.BlockSpec((1, D_OUT), lambda i, j: (0, 0)),
        ],
        out_specs=pl.BlockSpec((TPW, D_OUT), lambda i, j: (i, 0)),
        out_shape=jax.ShapeDtypeStruct((N_TOK, D_OUT), jnp.float32),
        compiler_params=pltpu.CompilerParams(
            dimension_semantics=("parallel", "arbitrary"),
        ),
    )(feat_g, thr3, wd_r, bdec2)


def kernel(x, W_enc, b_enc, threshold, W_dec, b_dec):
    B, S, _ = x.shape
    x2 = x.reshape(B * S, D_IN)
    beff = (b_enc - threshold).reshape(H, 1)
    bdec2 = b_dec.reshape(1, D_OUT)
    wd_r = W_dec.T.reshape(NHB, HB, D_OUT).astype(jnp.bfloat16)
    feat_g = _tc_encode(x2.T, beff, W_enc)
    thr = _sc_select(feat_g)
    out2 = _tc_decode(feat_g, thr.reshape(NW, 1, TPW), wd_r, bdec2)
    return out2.reshape(B, S, D_OUT)


# speedup vs baseline: 3.9155x; 1.0066x over previous
import functools

import jax
import jax.numpy as jnp
from jax import lax
from jax.experimental import pallas as pl
from jax.experimental.pallas import tpu as pltpu
from jax.experimental.pallas import tpu_sc as plsc

D_IN, H, D_OUT, K = 768, 16384, 768, 64
HB = 1024
NHB = H // HB
N_TOK = 4096
_INF_BITS = 0x7F800000

NW = 32
TPW = N_TOK // NW
NGRP = TPW // 16
SEG = 256
NSEG = H // SEG
CHUNK = 128
CAP = 384
TRIG = CAP - CHUNK - 1


def _enc_body(xt_ref, beff_ref, we_ref, feat_ref):
    pre = jnp.dot(we_ref[...], xt_ref[...], preferred_element_type=jnp.float32)
    feat_ref[...] = jnp.maximum(pre + beff_ref[...], 0.0)[None]


@jax.jit
def _tc_encode(xt, beff, W_enc):
    return pl.pallas_call(
        _enc_body,
        grid=(NW, NHB),
        in_specs=[
            pl.BlockSpec((D_IN, TPW), lambda i, j: (0, i)),
            pl.BlockSpec((HB, 1), lambda i, j: (j, 0)),
            pl.BlockSpec((HB, D_IN), lambda i, j: (j, 0)),
        ],
        out_specs=pl.BlockSpec((1, HB, TPW), lambda i, j: (i, j, 0)),
        out_shape=jax.ShapeDtypeStruct((NW, H, TPW), jnp.float32),
        compiler_params=pltpu.CompilerParams(
            dimension_semantics=("parallel", "arbitrary"),
        ),
    )(xt, beff, W_enc)


def _sc_body(feat_hbm, thr_hbm, segA, segB, cand, thrbuf, sem0, sem1):
    c_id = lax.axis_index("c")
    s_id = lax.axis_index("s")
    wid = s_id * 2 + c_id
    base = wid * TPW
    iota = lax.iota(jnp.int32, 16)
    zf = jnp.zeros((16,), jnp.float32)
    zi = jnp.zeros((16,), jnp.int32)

    def seg_copy(s, buf, sem):
        return pltpu.make_async_copy(
            feat_hbm.at[wid, pl.ds(s * SEG, SEG), :], buf, sem)

    def bisect(g, th, curs, iters):
        cbase = g * CAP
        maxc = jnp.max(curs)
        lo = plsc.bitcast(th, jnp.int32)
        hi = jnp.full((16,), _INF_BITS, jnp.int32)

        nblk8 = (maxc + 7) >> 3

        def bit(_, c2):
            lo, hi = c2
            mid = lo + ((hi - lo) >> 1)
            t = plsc.bitcast(mid, jnp.float32)

            def cnt(s, accs):
                a0, a1 = accs
                vv0 = cand[pl.ds((cbase + s) * 16, 16)]
                vv1 = cand[pl.ds((cbase + s + 1) * 16, 16)]
                return (a0 + jnp.where(vv0 > t, 1, 0),
                        a1 + jnp.where(vv1 > t, 1, 0))

            a0, a1 = plsc.parallel_loop(
                0, nblk8 * 8, step=2, unroll=4, carry=(zi, zi))(cnt)
            big = (a0 + a1) >= K
            return jnp.where(big, mid, lo), jnp.where(big, hi, mid)

        lo, _ = lax.fori_loop(0, iters, bit, (lo, hi))
        return lo

    def make_rebuild(g):
        cbase = g * CAP

        def rebuild(th_curs):
            th, curs = th_curs
            maxc = jnp.max(curs)
            tnew = plsc.bitcast(bisect(g, th, curs, 12), jnp.float32)

            def comp(s, nc):
                for u in range(4):
                    vv = cand[pl.ds((cbase + s * 4 + u) * 16, 16)]
                    cand[pl.ds((cbase + s * 4 + u) * 16, 16)] = zf
                    mm = vv > tnew
                    plsc.store_scatter(cand, [(cbase + nc) * 16 + iota],
                                       vv, mask=mm)
                    nc = nc + jnp.where(mm, 1, 0)
                return nc

            ncur = lax.fori_loop(0, (maxc + 3) >> 2, comp, zi)
            return tnew, ncur

        return rebuild

    UNROLL = 8

    def _step2(buf, idx, g, th, curs):
        v1 = buf[idx, pl.ds(g * 16, 16)]
        v2 = buf[idx + 1, pl.ds(g * 16, 16)]
        m1 = v1 > th
        m2 = v2 > th
        w1 = jnp.where(m1, 1, 0)
        w2 = jnp.where(m2, 1, 0)
        lim = (g + 1) * CAP * 16 - 1
        base = g * CAP * 16 + iota
        pos1 = jnp.minimum(curs * 16 + base, lim)
        pos2 = jnp.minimum((curs + w1) * 16 + base, lim)
        plsc.store_scatter(cand, [pos1], v1, mask=m1)
        plsc.store_scatter(cand, [pos2], v2, mask=m2)
        return curs + (w1 + w2)

    def make_pass2(ga, gb, buf):
        reb_a = make_rebuild(ga)
        reb_b = make_rebuild(gb)

        def chunk(c, carry2):
            def step(i, c3):
                (tha, cua), (thb, cub) = c3
                idx = c * CHUNK + i
                cua = _step2(buf, idx, ga, tha, cua)
                cub = _step2(buf, idx, gb, thb, cub)
                return (tha, cua), (thb, cub)

            sa, sb = plsc.parallel_loop(
                0, CHUNK, step=2, unroll=UNROLL // 2, carry=carry2)(step)
            sa = lax.cond(jnp.max(sa[1]) > TRIG, reb_a, lambda tc: tc, sa)
            sb = lax.cond(jnp.max(sb[1]) > TRIG, reb_b, lambda tc: tc, sb)
            return sa, sb

        return chunk

    def zero_cand():
        def zblk(s, x):
            cand[pl.ds(s * 16, 16)] = zf
            return x

        plsc.parallel_loop(0, NGRP * CAP, step=1, unroll=8,
                           carry=jnp.int32(0))(zblk)

    def run_seg(buf, states):
        out = list(states)
        for ga in range(0, NGRP, 2):
            sa, sb = lax.fori_loop(0, SEG // CHUNK, make_pass2(ga, ga + 1, buf),
                                   (states[ga], states[ga + 1]))
            out[ga], out[ga + 1] = sa, sb
        return tuple(out)

    zero_cand()
    states = tuple((zf, zi) for _ in range(NGRP))
    seg_copy(0, segA, sem0).start()

    def pair(p, states):
        seg_copy(2 * p, segA, sem0).wait()

        @pl.when(2 * p + 1 < NSEG)
        def _():
            seg_copy(2 * p + 1, segB, sem1).start()

        states = run_seg(segA, states)
        seg_copy(2 * p + 1, segB, sem1).wait()

        @pl.when(2 * p + 2 < NSEG)
        def _():
            seg_copy(2 * p + 2, segA, sem0).start()

        return run_seg(segB, states)

    states = lax.fori_loop(0, NSEG // 2, pair, states)
    for g in range(NGRP):
        th, curs = states[g]
        thrbuf[0:16] = plsc.bitcast(bisect(g, th, curs, 31), jnp.float32)
        pltpu.sync_copy(thrbuf, thr_hbm.at[pl.ds(base + g * 16, 16)])


_sc_select = pl.kernel(
    _sc_body,
    out_type=jax.ShapeDtypeStruct((N_TOK,), jnp.float32),
    mesh=plsc.VectorSubcoreMesh(core_axis_name="c", subcore_axis_name="s"),
    scratch_types=[
        pltpu.VMEM((SEG, TPW), jnp.float32),
        pltpu.VMEM((SEG, TPW), jnp.float32),
        pltpu.VMEM((NGRP * CAP * 16,), jnp.float32),
        pltpu.VMEM((16,), jnp.float32),
        pltpu.SemaphoreType.DMA,
        pltpu.SemaphoreType.DMA,
    ],
    compiler_params=pltpu.CompilerParams(needs_layout_passes=False),
)


def _dec_body(feat_ref, thr_ref, wd_ref, bdec_ref, out_ref):
    j = pl.program_id(1)

    @pl.when(j == 0)
    def _():
        out_ref[...] = jnp.broadcast_to(bdec_ref[...], (TPW, D_OUT))

    ft = feat_ref[0]
    m = jnp.where(ft > thr_ref[0], ft, 0.0).astype(jnp.bfloat16)
    out_ref[...] += lax.dot_general(
        m, wd_ref[0], (((0,), (0,)), ((), ())),
        preferred_element_type=jnp.float32)


@jax.jit
def _tc_decode(feat_g, thr3, wd_r, bdec2):
    return pl.pallas_call(
        _dec_body,
        grid=(NW, NHB),
        in_specs=[
            pl.BlockSpec((1, HB, TPW), lambda i, j: (i, j, 0)),
            pl.BlockSpec((1, 1, TPW), lambda i, j: (i, 0, 0)),
            pl.BlockSpec((1, HB, D_OUT), lambda i, j: (j, 0, 0)),
            pl.BlockSpec((1, D_OUT), lambda i, j: (0, 0)),
        ],
        out_specs=pl.BlockSpec((TPW, D_OUT), lambda i, j: (i, 0)),
        out_shape=jax.ShapeDtypeStruct((N_TOK, D_OUT), jnp.float32),
        compiler_params=pltpu.CompilerParams(
            dimension_semantics=("parallel", "arbitrary"),
        ),
    )(feat_g, thr3, wd_r, bdec2)


def kernel(x, W_enc, b_enc, threshold, W_dec, b_dec):
    B, S, _ = x.shape
    x2 = x.reshape(B * S, D_IN)
    beff = (b_enc - threshold).reshape(H, 1)
    bdec2 = b_dec.reshape(1, D_OUT)
    wd_r = W_dec.T.reshape(NHB, HB, D_OUT).astype(jnp.bfloat16)
    feat_g = _tc_encode(x2.T, beff, W_enc)
    thr = _sc_select(feat_g)
    out2 = _tc_decode(feat_g, thr.reshape(NW, 1, TPW), wd_r, bdec2)
    return out2.reshape(B, S, D_OUT)


# speedup vs baseline: 4.8478x; 1.2381x over previous
import functools

import jax
import jax.numpy as jnp
from jax import lax
from jax.experimental import pallas as pl
from jax.experimental.pallas import tpu as pltpu
from jax.experimental.pallas import tpu_sc as plsc

D_IN, H, D_OUT, K = 768, 16384, 768, 64
HB = 1024
NHB = H // HB
N_TOK = 4096
_INF_BITS = 0x7F800000

NW = 32
TPW = N_TOK // NW
NGRP = TPW // 16
SEG = 256
NSEG = H // SEG
CHUNK = 128
CAP = 384
TRIG = CAP - CHUNK - 1


def _enc_body(xt_ref, beff_ref, we_ref, feat_ref):
    pre = jnp.dot(we_ref[...], xt_ref[...], preferred_element_type=jnp.float32)
    feat_ref[...] = jnp.maximum(pre + beff_ref[...], 0.0)[None]


@jax.jit
def _tc_encode(xt, beff, W_enc):
    return pl.pallas_call(
        _enc_body,
        grid=(NW, NHB),
        in_specs=[
            pl.BlockSpec((D_IN, TPW), lambda i, j: (0, i)),
            pl.BlockSpec((HB, 1), lambda i, j: (j, 0)),
            pl.BlockSpec((HB, D_IN), lambda i, j: (j, 0)),
        ],
        out_specs=pl.BlockSpec((1, HB, TPW), lambda i, j: (i, j, 0)),
        out_shape=jax.ShapeDtypeStruct((NW, H, TPW), jnp.float32),
        compiler_params=pltpu.CompilerParams(
            dimension_semantics=("parallel", "arbitrary"),
        ),
    )(xt, beff, W_enc)


def _sc_body(feat_hbm, thr_hbm, segA, segB, cand, thrbuf, sem0, sem1):
    c_id = lax.axis_index("c")
    s_id = lax.axis_index("s")
    wid = s_id * 2 + c_id
    base = wid * TPW
    iota = lax.iota(jnp.int32, 16)
    zf = jnp.zeros((16,), jnp.float32)
    zi = jnp.zeros((16,), jnp.int32)

    def seg_copy(s, buf, sem):
        return pltpu.make_async_copy(
            feat_hbm.at[wid, pl.ds(s * SEG, SEG), :], buf, sem)

    def bisect(g, th, curs, iters):
        cbase = g * CAP
        maxc = jnp.max(curs)
        lo = plsc.bitcast(th, jnp.int32)
        hi = jnp.full((16,), _INF_BITS, jnp.int32)

        nblk8 = (maxc + 7) >> 3

        def bit(_, c2):
            lo, hi = c2
            mid = lo + ((hi - lo) >> 1)
            t = plsc.bitcast(mid, jnp.float32)

            def cnt(s, accs):
                a0, a1 = accs
                vv0 = cand[pl.ds((cbase + s) * 16, 16)]
                vv1 = cand[pl.ds((cbase + s + 1) * 16, 16)]
                return (a0 + jnp.where(vv0 > t, 1, 0),
                        a1 + jnp.where(vv1 > t, 1, 0))

            a0, a1 = plsc.parallel_loop(
                0, nblk8 * 8, step=2, unroll=4, carry=(zi, zi))(cnt)
            big = (a0 + a1) >= K
            return jnp.where(big, mid, lo), jnp.where(big, hi, mid)

        lo, _ = lax.fori_loop(0, iters, bit, (lo, hi))
        return lo

    def make_rebuild(g):
        cbase = g * CAP

        def rebuild(th_curs):
            th, curs = th_curs
            maxc = jnp.max(curs)
            tnew = plsc.bitcast(bisect(g, th, curs, 12), jnp.float32)

            def comp(s, nc):
                for u in range(4):
                    vv = cand[pl.ds((cbase + s * 4 + u) * 16, 16)]
                    cand[pl.ds((cbase + s * 4 + u) * 16, 16)] = zf
                    mm = vv > tnew
                    plsc.store_scatter(cand, [(cbase + nc) * 16 + iota],
                                       vv, mask=mm)
                    nc = nc + jnp.where(mm, 1, 0)
                return nc

            ncur = lax.fori_loop(0, (maxc + 3) >> 2, comp, zi)
            return tnew, ncur

        return rebuild

    UNROLL = 8

    def _step2(buf, idx, g, th, curs):
        v1 = buf[idx, pl.ds(g * 16, 16)]
        v2 = buf[idx + 1, pl.ds(g * 16, 16)]
        m1 = v1 > th
        m2 = v2 > th
        w1 = jnp.where(m1, 1, 0)
        w2 = jnp.where(m2, 1, 0)
        lim = (g + 1) * CAP * 16 - 1
        base = g * CAP * 16 + iota
        pos1 = jnp.minimum(curs * 16 + base, lim)
        pos2 = jnp.minimum((curs + w1) * 16 + base, lim)
        plsc.store_scatter(cand, [pos1], v1, mask=m1)
        plsc.store_scatter(cand, [pos2], v2, mask=m2)
        return curs + (w1 + w2)

    def make_pass2(ga, gb, buf):
        reb_a = make_rebuild(ga)
        reb_b = make_rebuild(gb)

        def chunk(c, carry2):
            def step(i, c3):
                (tha, cua), (thb, cub) = c3
                idx = c * CHUNK + i
                cua = _step2(buf, idx, ga, tha, cua)
                cub = _step2(buf, idx, gb, thb, cub)
                return (tha, cua), (thb, cub)

            sa, sb = plsc.parallel_loop(
                0, CHUNK, step=2, unroll=UNROLL, carry=carry2)(step)
            sa = lax.cond(jnp.max(sa[1]) > TRIG, reb_a, lambda tc: tc, sa)
            sb = lax.cond(jnp.max(sb[1]) > TRIG, reb_b, lambda tc: tc, sb)
            return sa, sb

        return chunk

    def zero_cand():
        def zblk(s, x):
            cand[pl.ds(s * 16, 16)] = zf
            return x

        plsc.parallel_loop(0, NGRP * CAP, step=1, unroll=8,
                           carry=jnp.int32(0))(zblk)

    def run_seg(buf, states):
        out = list(states)
        for ga in range(0, NGRP, 2):
            sa, sb = lax.fori_loop(0, SEG // CHUNK, make_pass2(ga, ga + 1, buf),
                                   (states[ga], states[ga + 1]))
            out[ga], out[ga + 1] = sa, sb
        return tuple(out)

    zero_cand()
    states = tuple((zf, zi) for _ in range(NGRP))
    seg_copy(0, segA, sem0).start()

    def pair(p, states):
        seg_copy(2 * p, segA, sem0).wait()

        @pl.when(2 * p + 1 < NSEG)
        def _():
            seg_copy(2 * p + 1, segB, sem1).start()

        states = run_seg(segA, states)
        seg_copy(2 * p + 1, segB, sem1).wait()

        @pl.when(2 * p + 2 < NSEG)
        def _():
            seg_copy(2 * p + 2, segA, sem0).start()

        return run_seg(segB, states)

    states = lax.fori_loop(0, NSEG // 2, pair, states)
    for g in range(NGRP):
        th, curs = states[g]
        thrbuf[0:16] = plsc.bitcast(bisect(g, th, curs, 31), jnp.float32)
        pltpu.sync_copy(thrbuf, thr_hbm.at[pl.ds(base + g * 16, 16)])


_sc_select = pl.kernel(
    _sc_body,
    out_type=jax.ShapeDtypeStruct((N_TOK,), jnp.float32),
    mesh=plsc.VectorSubcoreMesh(core_axis_name="c", subcore_axis_name="s"),
    scratch_types=[
        pltpu.VMEM((SEG, TPW), jnp.float32),
        pltpu.VMEM((SEG, TPW), jnp.float32),
        pltpu.VMEM((NGRP * CAP * 16,), jnp.float32),
        pltpu.VMEM((16,), jnp.float32),
        pltpu.SemaphoreType.DMA,
        pltpu.SemaphoreType.DMA,
    ],
    compiler_params=pltpu.CompilerParams(needs_layout_passes=False),
)


def _dec_body(feat_ref, thr_ref, wd_ref, bdec_ref, out_ref):
    j = pl.program_id(1)

    @pl.when(j == 0)
    def _():
        out_ref[...] = jnp.broadcast_to(bdec_ref[...], (TPW, D_OUT))

    ft = feat_ref[0]
    m = jnp.where(ft > thr_ref[0], ft, 0.0).astype(jnp.bfloat16)
    out_ref[...] += lax.dot_general(
        m, wd_ref[0], (((0,), (0,)), ((), ())),
        preferred_element_type=jnp.float32)


@jax.jit
def _tc_decode(feat_g, thr3, wd_r, bdec2):
    return pl.pallas_call(
        _dec_body,
        grid=(NW, NHB),
        in_specs=[
            pl.BlockSpec((1, HB, TPW), lambda i, j: (i, j, 0)),
            pl.BlockSpec((1, 1, TPW), lambda i, j: (i, 0, 0)),
            pl.BlockSpec((1, HB, D_OUT), lambda i, j: (j, 0, 0)),
            pl.BlockSpec((1, D_OUT), lambda i, j: (0, 0)),
        ],
        out_specs=pl.BlockSpec((TPW, D_OUT), lambda i, j: (i, 0)),
        out_shape=jax.ShapeDtypeStruct((N_TOK, D_OUT), jnp.float32),
        compiler_params=pltpu.CompilerParams(
            dimension_semantics=("parallel", "arbitrary"),
        ),
    )(feat_g, thr3, wd_r, bdec2)


def kernel(x, W_enc, b_enc, threshold, W_dec, b_dec):
    B, S, _ = x.shape
    x2 = x.reshape(B * S, D_IN)
    beff = (b_enc - threshold).reshape(H, 1)
    bdec2 = b_dec.reshape(1, D_OUT)
    wd_r = W_dec.T.reshape(NHB, HB, D_OUT).astype(jnp.bfloat16)
    feat_g = _tc_encode(x2.T, beff, W_enc)
    thr = _sc_select(feat_g)
    out2 = _tc_decode(feat_g, thr.reshape(NW, 1, TPW), wd_r, bdec2)
    return out2.reshape(B, S, D_OUT)
